# baseline (speedup 1.0000x reference)
import functools

import jax
import jax.numpy as jnp
from jax.experimental import pallas as pl
from jax.experimental.pallas import tpu as pltpu


def _fused_body(p_ref, w1_ref, b1_ref, w2f_ref, b2_ref, w3f_ref, b3_ref,
                cb_ref, cb2_ref, out_ref, *, T3, C, K, NB):
    f32 = jnp.float32
    zrow = jnp.zeros((1, C), f32)
    for bi in range(NB):

        hg = jnp.dot(p_ref[bi], w1_ref[...], preferred_element_type=f32)
        hg = jnp.maximum(hg + b1_ref[...], 0.0)
        p0 = hg[0 * T3:1 * T3]
        p1 = hg[1 * T3:2 * T3]
        p2 = hg[2 * T3:3 * T3]
        p3 = hg[3 * T3:4 * T3]
        p3_r = jnp.concatenate([zrow, p3[:-1, :]], axis=0)
        p0_l = jnp.concatenate([p0[1:, :], zrow], axis=0)

        pat2 = jnp.concatenate(
            [jnp.concatenate([p3_r, p0, p1, p2], axis=1),
             jnp.concatenate([p1, p2, p3, p0_l], axis=1)], axis=0)
        h2 = jnp.dot(pat2, w2f_ref[...], preferred_element_type=f32)
        h2 = jnp.maximum(h2 + b2_ref[...], 0.0)
        he = h2[:T3]
        ho = h2[T3:]
        ho_r = jnp.concatenate([zrow, ho[:-1, :]], axis=0)
        he_l = jnp.concatenate([he[1:, :], zrow], axis=0)

        pat3 = jnp.concatenate([ho_r, he, ho, he_l], axis=1)
        z = jnp.dot(pat3, w3f_ref[...], preferred_element_type=f32)
        z = z + b3_ref[...]

        cb = cb_ref[...]
        zc = jax.lax.dot_general(
            z, cb, (((1,), (1,)), ((), ())),
            preferred_element_type=f32)
        z2 = jnp.sum(z * z, axis=1, keepdims=True)
        d = z2 - 2.0 * zc + cb2_ref[...]
        minv = jnp.min(d, axis=1, keepdims=True)
        lane = jax.lax.broadcasted_iota(jnp.int32, (T3, K), 1)
        idx = jnp.min(jnp.where(d <= minv, lane, K), axis=1, keepdims=True)
        onehot = (lane == idx).astype(f32)
        qt = jax.lax.dot_general(
            cb, onehot, (((0,), (1,)), ((), ())),
            preferred_element_type=f32,
            precision=jax.lax.Precision.HIGHEST)
        out_ref[bi] = qt


def kernel(x, w1, b1, w2, b2, w3, b3, codebook):
    B, _, T = x.shape
    C = w1.shape[0]
    K = codebook.shape[0]
    T1, T3 = T // 2, T // 8

    xp = jnp.pad(x[:, 0, :], ((0, 0), (1, 1)))
    patches = jnp.stack([xp[:, k::2][:, :T1] for k in range(4)], axis=-1)
    patches = jnp.concatenate([patches[:, p::4, :] for p in range(4)],
                              axis=1)

    w1r = jnp.transpose(w1[:, 0, :])
    w2f = jnp.transpose(w2, (2, 1, 0)).reshape(4 * C, C)
    w3f = jnp.transpose(w3, (2, 1, 0)).reshape(4 * C, C)
    cb2 = jnp.sum(codebook * codebook, axis=1)[None, :]

    NB = 4
    body = functools.partial(_fused_body, T3=T3, C=C, K=K, NB=NB)
    return pl.pallas_call(
        body,
        grid=(B // NB,),
        in_specs=[
            pl.BlockSpec((4, T1, 4), lambda b: (b, 0, 0)),
            pl.BlockSpec((4, C), lambda b: (0, 0)),
            pl.BlockSpec((1, C), lambda b: (0, 0)),
            pl.BlockSpec((4 * C, C), lambda b: (0, 0)),
            pl.BlockSpec((1, C), lambda b: (0, 0)),
            pl.BlockSpec((4 * C, C), lambda b: (0, 0)),
            pl.BlockSpec((1, C), lambda b: (0, 0)),
            pl.BlockSpec((K, C), lambda b: (0, 0)),
            pl.BlockSpec((1, K), lambda b: (0, 0)),
        ],
        out_specs=pl.BlockSpec((4, C, T3), lambda b: (b, 0, 0)),
        out_shape=jax.ShapeDtypeStruct((B, C, T3), jnp.float32),
        compiler_params=pltpu.CompilerParams(
            dimension_semantics=("parallel",)),
    )(patches, w1r, b1[None, :], w2f, b2[None, :], w3f, b3[None, :],
      codebook, cb2)


# speedup vs baseline: 1.1099x; 1.1099x over previous
import functools

import jax
import jax.numpy as jnp
from jax.experimental import pallas as pl
from jax.experimental.pallas import tpu as pltpu


def _fused_body(p_ref, w1_ref, b1_ref, w2f_ref, b2_ref, w3f_ref, b3_ref,
                cb_ref, cb2_ref, out_ref, *, T3, C, K):
    f32 = jnp.float32
    zrow = jnp.zeros((1, C), f32)

    hg = jnp.dot(p_ref[0], w1_ref[...], preferred_element_type=f32)
    hg = jnp.maximum(hg + b1_ref[...], 0.0)
    p0 = hg[0 * T3:1 * T3]
    p1 = hg[1 * T3:2 * T3]
    p2 = hg[2 * T3:3 * T3]
    p3 = hg[3 * T3:4 * T3]
    p3_r = jnp.concatenate([zrow, p3[:-1, :]], axis=0)
    p0_l = jnp.concatenate([p0[1:, :], zrow], axis=0)

    pat2 = jnp.concatenate(
        [jnp.concatenate([p3_r, p0, p1, p2], axis=1),
         jnp.concatenate([p1, p2, p3, p0_l], axis=1)], axis=0)
    h2 = jnp.dot(pat2, w2f_ref[...], preferred_element_type=f32)
    h2 = jnp.maximum(h2 + b2_ref[...], 0.0)
    he = h2[:T3]
    ho = h2[T3:]
    ho_r = jnp.concatenate([zrow, ho[:-1, :]], axis=0)
    he_l = jnp.concatenate([he[1:, :], zrow], axis=0)

    pat3 = jnp.concatenate([ho_r, he, ho, he_l], axis=1)
    z = jnp.dot(pat3, w3f_ref[...], preferred_element_type=f32)
    z = z + b3_ref[...]

    cb = cb_ref[...]
    zc = jax.lax.dot_general(
        z, cb, (((1,), (1,)), ((), ())),
        preferred_element_type=f32)
    z2 = jnp.sum(z * z, axis=1, keepdims=True)
    d = z2 - 2.0 * zc + cb2_ref[...]
    minv = jnp.min(d, axis=1, keepdims=True)
    lane = jax.lax.broadcasted_iota(jnp.int32, (T3, K), 1)
    idx = jnp.min(jnp.where(d <= minv, lane, K), axis=1, keepdims=True)
    onehot = (lane == idx).astype(f32)
    qt = jax.lax.dot_general(
        cb, onehot, (((0,), (1,)), ((), ())),
        preferred_element_type=f32,
        precision=jax.lax.Precision.HIGHEST)
    out_ref[0] = qt


def kernel(x, w1, b1, w2, b2, w3, b3, codebook):
    B, _, T = x.shape
    C = w1.shape[0]
    K = codebook.shape[0]
    T1, T3 = T // 2, T // 8

    xp = jnp.pad(x[:, 0, :], ((0, 0), (1, 7)))
    x8 = xp.reshape(B, T // 8 + 1, 8)
    nt = T // 8
    blocks = [x8[:, :nt, 0:4], x8[:, :nt, 2:6], x8[:, :nt, 4:8],
              jnp.concatenate([x8[:, :nt, 6:8], x8[:, 1:, 0:2]], axis=2)]
    patches = jnp.concatenate(blocks, axis=1)

    w1r = jnp.transpose(w1[:, 0, :])
    w2f = jnp.transpose(w2, (2, 1, 0)).reshape(4 * C, C)
    w3f = jnp.transpose(w3, (2, 1, 0)).reshape(4 * C, C)
    cb2 = jnp.sum(codebook * codebook, axis=1)[None, :]

    body = functools.partial(_fused_body, T3=T3, C=C, K=K)
    return pl.pallas_call(
        body,
        grid=(B,),
        in_specs=[
            pl.BlockSpec((1, T1, 4), lambda b: (b, 0, 0)),
            pl.BlockSpec((4, C), lambda b: (0, 0)),
            pl.BlockSpec((1, C), lambda b: (0, 0)),
            pl.BlockSpec((4 * C, C), lambda b: (0, 0)),
            pl.BlockSpec((1, C), lambda b: (0, 0)),
            pl.BlockSpec((4 * C, C), lambda b: (0, 0)),
            pl.BlockSpec((1, C), lambda b: (0, 0)),
            pl.BlockSpec((K, C), lambda b: (0, 0)),
            pl.BlockSpec((1, K), lambda b: (0, 0)),
        ],
        out_specs=pl.BlockSpec((1, C, T3), lambda b: (b, 0, 0)),
        out_shape=jax.ShapeDtypeStruct((B, C, T3), jnp.float32),
        compiler_params=pltpu.CompilerParams(
            dimension_semantics=("parallel",)),
    )(patches, w1r, b1[None, :], w2f, b2[None, :], w3f, b3[None, :],
      codebook, cb2)


# speedup vs baseline: 1.5844x; 1.4275x over previous
import functools

import jax
import jax.numpy as jnp
from jax.experimental import pallas as pl
from jax.experimental.pallas import tpu as pltpu


def _fused_body(p_ref, w1_ref, b1_ref, w2f_ref, b2_ref, w3f_ref, b3_ref,
                cb_ref, cb2_ref, out_ref, *, T3, C, K):
    f32 = jnp.float32
    zrow = jnp.zeros((1, C), f32)

    yv = p_ref[0]
    p0, p1, p2, p3 = (
        jnp.maximum(jnp.dot(yv, w1_ref[p], preferred_element_type=f32)
                    + b1_ref[...], 0.0)
        for p in range(4))
    p3_r = jnp.concatenate([zrow, p3[:-1, :]], axis=0)
    p0_l = jnp.concatenate([p0[1:, :], zrow], axis=0)

    pat2 = jnp.concatenate(
        [jnp.concatenate([p3_r, p0, p1, p2], axis=1),
         jnp.concatenate([p1, p2, p3, p0_l], axis=1)], axis=0)
    h2 = jnp.dot(pat2, w2f_ref[...], preferred_element_type=f32)
    h2 = jnp.maximum(h2 + b2_ref[...], 0.0)
    he = h2[:T3]
    ho = h2[T3:]
    ho_r = jnp.concatenate([zrow, ho[:-1, :]], axis=0)
    he_l = jnp.concatenate([he[1:, :], zrow], axis=0)

    pat3 = jnp.concatenate([ho_r, he, ho, he_l], axis=1)
    z = jnp.dot(pat3, w3f_ref[...], preferred_element_type=f32)
    z = z + b3_ref[...]

    cb = cb_ref[...]
    zc = jax.lax.dot_general(
        z, cb, (((1,), (1,)), ((), ())),
        preferred_element_type=f32)
    z2 = jnp.sum(z * z, axis=1, keepdims=True)
    d = z2 - 2.0 * zc + cb2_ref[...]
    minv = jnp.min(d, axis=1, keepdims=True)
    lane = jax.lax.broadcasted_iota(jnp.int32, (T3, K), 1)
    idx = jnp.min(jnp.where(d <= minv, lane, K), axis=1, keepdims=True)
    onehot = (lane == idx).astype(f32)
    qt = jax.lax.dot_general(
        cb, onehot, (((0,), (1,)), ((), ())),
        preferred_element_type=f32,
        precision=jax.lax.Precision.HIGHEST)
    out_ref[0] = qt


def kernel(x, w1, b1, w2, b2, w3, b3, codebook):
    B, _, T = x.shape
    C = w1.shape[0]
    K = codebook.shape[0]
    T3 = T // 8

    xp = jnp.pad(x[:, 0, :], ((0, 0), (1, 7)))
    nt = T // 8
    x8 = xp.reshape(B, nt + 1, 8)
    y = jnp.concatenate([x8[:, :nt], x8[:, 1:]], axis=2)

    w1r = jnp.transpose(w1[:, 0, :])
    w1s = jnp.stack([jnp.pad(w1r, ((2 * p, 12 - 2 * p), (0, 0)))
                     for p in range(4)])
    w2f = jnp.transpose(w2, (2, 1, 0)).reshape(4 * C, C)
    w3f = jnp.transpose(w3, (2, 1, 0)).reshape(4 * C, C)
    cb2 = jnp.sum(codebook * codebook, axis=1)[None, :]

    body = functools.partial(_fused_body, T3=T3, C=C, K=K)
    return pl.pallas_call(
        body,
        grid=(B,),
        in_specs=[
            pl.BlockSpec((1, T3, 16), lambda b: (b, 0, 0)),
            pl.BlockSpec((4, 16, C), lambda b: (0, 0, 0)),
            pl.BlockSpec((1, C), lambda b: (0, 0)),
            pl.BlockSpec((4 * C, C), lambda b: (0, 0)),
            pl.BlockSpec((1, C), lambda b: (0, 0)),
            pl.BlockSpec((4 * C, C), lambda b: (0, 0)),
            pl.BlockSpec((1, C), lambda b: (0, 0)),
            pl.BlockSpec((K, C), lambda b: (0, 0)),
            pl.BlockSpec((1, K), lambda b: (0, 0)),
        ],
        out_specs=pl.BlockSpec((1, C, T3), lambda b: (b, 0, 0)),
        out_shape=jax.ShapeDtypeStruct((B, C, T3), jnp.float32),
        compiler_params=pltpu.CompilerParams(
            dimension_semantics=("parallel",)),
    )(y, w1s, b1[None, :], w2f, b2[None, :], w3f, b3[None, :],
      codebook, cb2)


# speedup vs baseline: 1.7133x; 1.0814x over previous
import functools

import jax
import jax.numpy as jnp
from jax.experimental import pallas as pl
from jax.experimental.pallas import tpu as pltpu

_NB = 2


def _fused_body(p_ref, w1_ref, b1_ref, w2f_ref, b2_ref, w3f_ref, b3_ref,
                cb_ref, cb2_ref, out_ref, *, T3, C, K, NB):
    f32 = jnp.float32
    zrow = jnp.zeros((1, C), f32)
    R = NB * T3

    def shift_r(a):
        parts = []
        for b in range(NB):
            parts += [zrow, a[b * T3:(b + 1) * T3 - 1]]
        return jnp.concatenate(parts, axis=0)

    def shift_l(a):
        parts = []
        for b in range(NB):
            parts += [a[b * T3 + 1:(b + 1) * T3], zrow]
        return jnp.concatenate(parts, axis=0)

    yv = p_ref[...].reshape(R, 16)
    p0, p1, p2, p3 = (
        jnp.maximum(jnp.dot(yv, w1_ref[p], preferred_element_type=f32)
                    + b1_ref[...], 0.0)
        for p in range(4))
    p3_r = shift_r(p3)
    p0_l = shift_l(p0)

    pat2 = jnp.concatenate(
        [jnp.concatenate([p3_r, p0, p1, p2], axis=1),
         jnp.concatenate([p1, p2, p3, p0_l], axis=1)], axis=0)
    h2 = jnp.dot(pat2, w2f_ref[...], preferred_element_type=f32)
    h2 = jnp.maximum(h2 + b2_ref[...], 0.0)
    he = h2[:R]
    ho = h2[R:]
    ho_r = shift_r(ho)
    he_l = shift_l(he)

    pat3 = jnp.concatenate([ho_r, he, ho, he_l], axis=1)
    z = jnp.dot(pat3, w3f_ref[...], preferred_element_type=f32)
    z = z + b3_ref[...]

    cb = cb_ref[...]
    zc = jax.lax.dot_general(
        z, cb, (((1,), (1,)), ((), ())),
        preferred_element_type=f32)
    z2 = jnp.sum(z * z, axis=1, keepdims=True)
    d = z2 - 2.0 * zc + cb2_ref[...]
    minv = jnp.min(d, axis=1, keepdims=True)
    lane = jax.lax.broadcasted_iota(jnp.int32, (R, K), 1)
    idx = jnp.min(jnp.where(d <= minv, lane, K), axis=1, keepdims=True)
    onehot = (lane == idx).astype(f32)
    qt = jax.lax.dot_general(
        cb, onehot, (((0,), (1,)), ((), ())),
        preferred_element_type=f32,
        precision=jax.lax.Precision.HIGHEST)
    for b in range(NB):
        out_ref[b] = qt[:, b * T3:(b + 1) * T3]


def kernel(x, w1, b1, w2, b2, w3, b3, codebook):
    B, _, T = x.shape
    C = w1.shape[0]
    K = codebook.shape[0]
    T3 = T // 8

    xp = jnp.pad(x[:, 0, :], ((0, 0), (1, 7)))
    nt = T // 8
    x8 = xp.reshape(B, nt + 1, 8)
    y = jnp.concatenate([x8[:, :nt], x8[:, 1:]], axis=2)

    w1r = jnp.transpose(w1[:, 0, :])
    w1s = jnp.stack([jnp.pad(w1r, ((2 * p, 12 - 2 * p), (0, 0)))
                     for p in range(4)])
    w2f = jnp.transpose(w2, (2, 1, 0)).reshape(4 * C, C)
    w3f = jnp.transpose(w3, (2, 1, 0)).reshape(4 * C, C)
    cb2 = jnp.sum(codebook * codebook, axis=1)[None, :]

    body = functools.partial(_fused_body, T3=T3, C=C, K=K, NB=_NB)
    return pl.pallas_call(
        body,
        grid=(B // _NB,),
        in_specs=[
            pl.BlockSpec((_NB, T3, 16), lambda b: (b, 0, 0)),
            pl.BlockSpec((4, 16, C), lambda b: (0, 0, 0)),
            pl.BlockSpec((1, C), lambda b: (0, 0)),
            pl.BlockSpec((4 * C, C), lambda b: (0, 0)),
            pl.BlockSpec((1, C), lambda b: (0, 0)),
            pl.BlockSpec((4 * C, C), lambda b: (0, 0)),
            pl.BlockSpec((1, C), lambda b: (0, 0)),
            pl.BlockSpec((K, C), lambda b: (0, 0)),
            pl.BlockSpec((1, K), lambda b: (0, 0)),
        ],
        out_specs=pl.BlockSpec((_NB, C, T3), lambda b: (b, 0, 0)),
        out_shape=jax.ShapeDtypeStruct((B, C, T3), jnp.float32),
        compiler_params=pltpu.CompilerParams(
            dimension_semantics=("parallel",)),
    )(y, w1s, b1[None, :], w2f, b2[None, :], w3f, b3[None, :],
      codebook, cb2)


# speedup vs baseline: 1.7475x; 1.0200x over previous
import functools

import jax
import jax.numpy as jnp
from jax.experimental import pallas as pl
from jax.experimental.pallas import tpu as pltpu

_NB = 4


def _fused_body(p_ref, w1_ref, b1_ref, w2f_ref, b2_ref, w3f_ref, b3_ref,
                cb_ref, cb2_ref, out_ref, *, T3, C, K, NB):
    f32 = jnp.float32
    zrow = jnp.zeros((1, C), f32)
    R = NB * T3

    def shift_r(a):
        parts = []
        for b in range(NB):
            parts += [zrow, a[b * T3:(b + 1) * T3 - 1]]
        return jnp.concatenate(parts, axis=0)

    def shift_l(a):
        parts = []
        for b in range(NB):
            parts += [a[b * T3 + 1:(b + 1) * T3], zrow]
        return jnp.concatenate(parts, axis=0)

    yv = p_ref[...].reshape(R, 16)
    p0, p1, p2, p3 = (
        jnp.maximum(jnp.dot(yv, w1_ref[p], preferred_element_type=f32)
                    + b1_ref[...], 0.0)
        for p in range(4))
    p3_r = shift_r(p3)
    p0_l = shift_l(p0)

    pat2 = jnp.concatenate(
        [jnp.concatenate([p3_r, p0, p1, p2], axis=1),
         jnp.concatenate([p1, p2, p3, p0_l], axis=1)], axis=0)
    h2 = jnp.dot(pat2, w2f_ref[...], preferred_element_type=f32)
    h2 = jnp.maximum(h2 + b2_ref[...], 0.0)
    he = h2[:R]
    ho = h2[R:]
    ho_r = shift_r(ho)
    he_l = shift_l(he)

    pat3 = jnp.concatenate([ho_r, he, ho, he_l], axis=1)
    z = jnp.dot(pat3, w3f_ref[...], preferred_element_type=f32)
    z = z + b3_ref[...]

    cb = cb_ref[...]
    zc = jax.lax.dot_general(
        z, cb, (((1,), (1,)), ((), ())),
        preferred_element_type=f32)
    z2 = jnp.sum(z * z, axis=1, keepdims=True)
    d = z2 - 2.0 * zc + cb2_ref[...]
    minv = jnp.min(d, axis=1, keepdims=True)
    lane = jax.lax.broadcasted_iota(jnp.int32, (R, K), 1)
    idx = jnp.min(jnp.where(d <= minv, lane, K), axis=1, keepdims=True)
    onehot = (lane == idx).astype(f32)
    qt = jax.lax.dot_general(
        cb, onehot, (((0,), (1,)), ((), ())),
        preferred_element_type=f32,
        precision=jax.lax.Precision.HIGHEST)
    for b in range(NB):
        out_ref[b] = qt[:, b * T3:(b + 1) * T3]


def kernel(x, w1, b1, w2, b2, w3, b3, codebook):
    B, _, T = x.shape
    C = w1.shape[0]
    K = codebook.shape[0]
    T3 = T // 8

    xp = jnp.pad(x[:, 0, :], ((0, 0), (1, 7)))
    nt = T // 8
    x8 = xp.reshape(B, nt + 1, 8)
    y = jnp.concatenate([x8[:, :nt], x8[:, 1:]], axis=2)

    w1r = jnp.transpose(w1[:, 0, :])
    w1s = jnp.stack([jnp.pad(w1r, ((2 * p, 12 - 2 * p), (0, 0)))
                     for p in range(4)])
    w2f = jnp.transpose(w2, (2, 1, 0)).reshape(4 * C, C)
    w3f = jnp.transpose(w3, (2, 1, 0)).reshape(4 * C, C)
    cb2 = jnp.sum(codebook * codebook, axis=1)[None, :]

    body = functools.partial(_fused_body, T3=T3, C=C, K=K, NB=_NB)
    return pl.pallas_call(
        body,
        grid=(B // _NB,),
        in_specs=[
            pl.BlockSpec((_NB, T3, 16), lambda b: (b, 0, 0)),
            pl.BlockSpec((4, 16, C), lambda b: (0, 0, 0)),
            pl.BlockSpec((1, C), lambda b: (0, 0)),
            pl.BlockSpec((4 * C, C), lambda b: (0, 0)),
            pl.BlockSpec((1, C), lambda b: (0, 0)),
            pl.BlockSpec((4 * C, C), lambda b: (0, 0)),
            pl.BlockSpec((1, C), lambda b: (0, 0)),
            pl.BlockSpec((K, C), lambda b: (0, 0)),
            pl.BlockSpec((1, K), lambda b: (0, 0)),
        ],
        out_specs=pl.BlockSpec((_NB, C, T3), lambda b: (b, 0, 0)),
        out_shape=jax.ShapeDtypeStruct((B, C, T3), jnp.float32),
        compiler_params=pltpu.CompilerParams(
            dimension_semantics=("parallel",)),
    )(y, w1s, b1[None, :], w2f, b2[None, :], w3f, b3[None, :],
      codebook, cb2)


# speedup vs baseline: 1.9523x; 1.1172x over previous
import functools

import jax
import jax.numpy as jnp
from jax.experimental import pallas as pl
from jax.experimental.pallas import tpu as pltpu

_NB = 4


def _fused_body(p_ref, w1_ref, b1_ref, w2f_ref, b2_ref, w3f_ref, b3_ref,
                cb_ref, cb2_ref, cb3_ref, out_ref, *, T3, C, K, NB):
    f32 = jnp.float32
    zrow = jnp.zeros((1, C), f32)
    R = NB * T3

    def shift_r(a):
        parts = []
        for b in range(NB):
            parts += [zrow, a[b * T3:(b + 1) * T3 - 1]]
        return jnp.concatenate(parts, axis=0)

    def shift_l(a):
        parts = []
        for b in range(NB):
            parts += [a[b * T3 + 1:(b + 1) * T3], zrow]
        return jnp.concatenate(parts, axis=0)

    yv = p_ref[...].reshape(R, 16)
    p0, p1, p2, p3 = (
        jnp.maximum(jnp.dot(yv, w1_ref[p], preferred_element_type=f32)
                    + b1_ref[...], 0.0)
        for p in range(4))
    p3_r = shift_r(p3)
    p0_l = shift_l(p0)

    pat2 = jnp.concatenate(
        [jnp.concatenate([p3_r, p0, p1, p2], axis=1),
         jnp.concatenate([p1, p2, p3, p0_l], axis=1)], axis=0)
    h2 = jnp.dot(pat2, w2f_ref[...], preferred_element_type=f32)
    h2 = jnp.maximum(h2 + b2_ref[...], 0.0)
    he = h2[:R]
    ho = h2[R:]
    ho_r = shift_r(ho)
    he_l = shift_l(he)

    pat3 = jnp.concatenate([ho_r, he, ho, he_l], axis=1)
    z = jnp.dot(pat3, w3f_ref[...], preferred_element_type=f32)
    z = z + b3_ref[...]

    cb = cb_ref[...]
    zc = jax.lax.dot_general(
        z, cb, (((1,), (1,)), ((), ())),
        preferred_element_type=f32)
    z2 = jnp.sum(z * z, axis=1, keepdims=True)
    d = z2 - 2.0 * zc + cb2_ref[...]
    minv = jnp.min(d, axis=1, keepdims=True)
    lane = jax.lax.broadcasted_iota(jnp.int32, (R, K), 1)
    idx = jnp.min(jnp.where(d <= minv, lane, K), axis=1, keepdims=True)
    onehot = (lane == idx).astype(jnp.bfloat16)
    dg = functools.partial(
        jax.lax.dot_general,
        dimension_numbers=(((0,), (1,)), ((), ())),
        preferred_element_type=f32)
    qt = ((dg(cb3_ref[0], onehot) + dg(cb3_ref[1], onehot))
          + dg(cb3_ref[2], onehot))
    for b in range(NB):
        out_ref[b] = qt[:, b * T3:(b + 1) * T3]


def kernel(x, w1, b1, w2, b2, w3, b3, codebook):
    B, _, T = x.shape
    C = w1.shape[0]
    K = codebook.shape[0]
    T3 = T // 8

    xp = jnp.pad(x[:, 0, :], ((0, 0), (1, 7)))
    nt = T // 8
    x8 = xp.reshape(B, nt + 1, 8)
    y = jnp.concatenate([x8[:, :nt], x8[:, 1:]], axis=2)

    w1r = jnp.transpose(w1[:, 0, :])
    w1s = jnp.stack([jnp.pad(w1r, ((2 * p, 12 - 2 * p), (0, 0)))
                     for p in range(4)])
    w2f = jnp.transpose(w2, (2, 1, 0)).reshape(4 * C, C)
    w3f = jnp.transpose(w3, (2, 1, 0)).reshape(4 * C, C)
    cb2 = jnp.sum(codebook * codebook, axis=1)[None, :]
    bf16 = jnp.bfloat16
    cb_hi = codebook.astype(bf16)
    r1 = codebook - cb_hi.astype(jnp.float32)
    cb_mid = r1.astype(bf16)
    cb_lo = (r1 - cb_mid.astype(jnp.float32)).astype(bf16)
    cb3 = jnp.stack([cb_hi, cb_mid, cb_lo])

    body = functools.partial(_fused_body, T3=T3, C=C, K=K, NB=_NB)
    return pl.pallas_call(
        body,
        grid=(B // _NB,),
        in_specs=[
            pl.BlockSpec((_NB, T3, 16), lambda b: (b, 0, 0)),
            pl.BlockSpec((4, 16, C), lambda b: (0, 0, 0)),
            pl.BlockSpec((1, C), lambda b: (0, 0)),
            pl.BlockSpec((4 * C, C), lambda b: (0, 0)),
            pl.BlockSpec((1, C), lambda b: (0, 0)),
            pl.BlockSpec((4 * C, C), lambda b: (0, 0)),
            pl.BlockSpec((1, C), lambda b: (0, 0)),
            pl.BlockSpec((K, C), lambda b: (0, 0)),
            pl.BlockSpec((1, K), lambda b: (0, 0)),
            pl.BlockSpec((3, K, C), lambda b: (0, 0, 0)),
        ],
        out_specs=pl.BlockSpec((_NB, C, T3), lambda b: (b, 0, 0)),
        out_shape=jax.ShapeDtypeStruct((B, C, T3), jnp.float32),
        compiler_params=pltpu.CompilerParams(
            dimension_semantics=("parallel",)),
    )(y, w1s, b1[None, :], w2f, b2[None, :], w3f, b3[None, :],
      codebook, cb2, cb3)


# speedup vs baseline: 1.9622x; 1.0050x over previous
import functools

import jax
import jax.numpy as jnp
from jax.experimental import pallas as pl
from jax.experimental.pallas import tpu as pltpu

_NB = 4


def _fused_body(p_ref, w1_ref, b1_ref, w2f_ref, b2_ref, w3f_ref, b3_ref,
                cb_ref, cb2_ref, cb3_ref, out_ref, *, T3, C, K, NB):
    f32 = jnp.float32
    zrow = jnp.zeros((1, C), f32)
    R = NB * T3

    def shift_r(a):
        parts = []
        for b in range(NB):
            parts += [zrow, a[b * T3:(b + 1) * T3 - 1]]
        return jnp.concatenate(parts, axis=0)

    def shift_l(a):
        parts = []
        for b in range(NB):
            parts += [a[b * T3 + 1:(b + 1) * T3], zrow]
        return jnp.concatenate(parts, axis=0)

    yv = p_ref[...].reshape(R, 16)
    p0, p1, p2, p3 = (
        jnp.maximum(jnp.dot(yv, w1_ref[p], preferred_element_type=f32)
                    + b1_ref[...], 0.0)
        for p in range(4))
    p3_r = shift_r(p3)
    p0_l = shift_l(p0)

    pat2 = jnp.concatenate(
        [jnp.concatenate([p3_r, p0, p1, p2], axis=1),
         jnp.concatenate([p1, p2, p3, p0_l], axis=1)], axis=0)
    h2 = jnp.dot(pat2, w2f_ref[...], preferred_element_type=f32)
    h2 = jnp.maximum(h2 + b2_ref[...], 0.0)
    he = h2[:R]
    ho = h2[R:]
    ho_r = shift_r(ho)
    he_l = shift_l(he)

    pat3 = jnp.concatenate([ho_r, he, ho, he_l], axis=1)
    z = jnp.dot(pat3, w3f_ref[...], preferred_element_type=f32)
    z = z + b3_ref[...]

    cb = cb_ref[...]
    zc = jax.lax.dot_general(
        z, cb, (((1,), (1,)), ((), ())),
        preferred_element_type=f32)
    z2 = jnp.sum(z * z, axis=1, keepdims=True)
    d = z2 - 2.0 * zc + cb2_ref[...]
    minv = jnp.min(d, axis=1, keepdims=True)
    lane = jax.lax.broadcasted_iota(jnp.int32, (R, K), 1)
    idx = jnp.min(jnp.where(d <= minv, lane, K), axis=1, keepdims=True)
    onehot = (lane == idx).astype(f32)
    dg = functools.partial(
        jax.lax.dot_general,
        dimension_numbers=(((0,), (1,)), ((), ())),
        preferred_element_type=f32)
    qt = ((dg(cb3_ref[0], onehot) + dg(cb3_ref[1], onehot))
          + dg(cb3_ref[2], onehot))
    for b in range(NB):
        out_ref[b] = qt[:, b * T3:(b + 1) * T3]


def kernel(x, w1, b1, w2, b2, w3, b3, codebook):
    B, _, T = x.shape
    C = w1.shape[0]
    K = codebook.shape[0]
    T3 = T // 8

    xp = jnp.pad(x[:, 0, :], ((0, 0), (1, 7)))
    nt = T // 8
    x8 = xp.reshape(B, nt + 1, 8)
    y = jnp.concatenate([x8[:, :nt], x8[:, 1:]], axis=2)

    w1r = jnp.transpose(w1[:, 0, :])
    w1s = jnp.stack([jnp.pad(w1r, ((2 * p, 12 - 2 * p), (0, 0)))
                     for p in range(4)])
    w2f = jnp.transpose(w2, (2, 1, 0)).reshape(4 * C, C)
    w3f = jnp.transpose(w3, (2, 1, 0)).reshape(4 * C, C)
    cb2 = jnp.sum(codebook * codebook, axis=1)[None, :]
    bf16 = jnp.bfloat16
    cb_hi = codebook.astype(bf16).astype(jnp.float32)
    r1 = codebook - cb_hi
    cb_mid = r1.astype(bf16).astype(jnp.float32)
    cb_lo = r1 - cb_mid
    cb3 = jnp.stack([cb_hi, cb_mid, cb_lo])

    body = functools.partial(_fused_body, T3=T3, C=C, K=K, NB=_NB)
    return pl.pallas_call(
        body,
        grid=(B // _NB,),
        in_specs=[
            pl.BlockSpec((_NB, T3, 16), lambda b: (b, 0, 0)),
            pl.BlockSpec((4, 16, C), lambda b: (0, 0, 0)),
            pl.BlockSpec((1, C), lambda b: (0, 0)),
            pl.BlockSpec((4 * C, C), lambda b: (0, 0)),
            pl.BlockSpec((1, C), lambda b: (0, 0)),
            pl.BlockSpec((4 * C, C), lambda b: (0, 0)),
            pl.BlockSpec((1, C), lambda b: (0, 0)),
            pl.BlockSpec((K, C), lambda b: (0, 0)),
            pl.BlockSpec((1, K), lambda b: (0, 0)),
            pl.BlockSpec((3, K, C), lambda b: (0, 0, 0)),
        ],
        out_specs=pl.BlockSpec((_NB, C, T3), lambda b: (b, 0, 0)),
        out_shape=jax.ShapeDtypeStruct((B, C, T3), jnp.float32),
        compiler_params=pltpu.CompilerParams(
            dimension_semantics=("parallel",)),
    )(y, w1s, b1[None, :], w2f, b2[None, :], w3f, b3[None, :],
      codebook, cb2, cb3)
